# baseline (speedup 1.0000x reference)
import functools

import jax
import jax.numpy as jnp
from jax import lax
from jax.experimental import pallas as pl
from jax.experimental.pallas import tpu as pltpu
from jax.experimental.pallas import tpu_sc as plsc

_V = 100000
_VP = 782 * 128
_E = 64
_B = 1024
_S = 200
_N = _B * _S

_NC = 2
_NS = 16
_PASSES = 2
_SG = _S // 8


def _pe_table_t():
    pos = jnp.arange(_S, dtype=jnp.float32)[:, None]
    denom = 10000.0 ** ((jnp.arange(_E) // 2).astype(jnp.float32) / _E)[None, :]
    ang = pos / denom
    pe = jnp.where((jnp.arange(_E) % 2)[None, :] == 0, jnp.sin(ang), jnp.cos(ang))
    return jnp.pad(pe.T, ((0, 0), (0, 1024 - _S))).reshape(_E, 8, 128)


def _sc_call(idx_flat, tp3, pe8):
    mesh = plsc.VectorSubcoreMesh(core_axis_name="c", subcore_axis_name="s")

    @functools.partial(
        pl.kernel,
        mesh=mesh,
        out_type=jax.ShapeDtypeStruct((_S, 8, 8, 8, 128), jnp.float32),
        compiler_params=pltpu.CompilerParams(needs_layout_passes=False),
        scratch_types=[
            pltpu.VMEM_SHARED((2, 2, 8, 8, 8, 128), jnp.float32),
            pltpu.VMEM((782, 128), jnp.float32),
            pltpu.VMEM((8 * _B,), jnp.int32),
            pltpu.VMEM((8, 128), jnp.float32),
            pltpu.VMEM((8, 128), jnp.float32),
            pltpu.VMEM((8, 128), jnp.float32),
            pltpu.SemaphoreType.DMA,
            pltpu.SemaphoreType.DMA,
            pltpu.SemaphoreType.DMA,
        ],
    )
    def k(idx_hbm, tp3_hbm, pe8_hbm, out_hbm, spo, row_v, idx_v,
          pe_v, or0, or1, os0, os1, dsem):
        sid = lax.axis_index("s")
        cid = lax.axis_index("c")
        grp_l = sid // 8
        sub = sid % 8
        orows = (or0, or1)
        osems = (os0, os1)

        for p in range(_PASSES):
            g = 4 * p + 2 * cid + grp_l
            e = 8 * g + sub
            pltpu.sync_copy(tp3_hbm.at[e], row_v)
            pltpu.sync_copy(pe8_hbm.at[e], pe_v)

            def sg_body(sg, carry):
                buf = sg & 1
                pltpu.sync_copy(idx_hbm.at[pl.ds(sg * (8 * _B), 8 * _B)],
                                idx_v)
                for sp in range(4):
                    for q in range(2):
                        si = 2 * sp + q
                        s = sg * 8 + si
                        orow = orows[q]
                        osem = osems[q]
                        spo_dst = spo.at[buf, grp_l, si, :, sub, :]
                        if sp > 0:
                            pltpu.make_async_copy(orow, spo_dst, osem).wait()
                        hi16 = jnp.full((16,), s // 128, dtype=jnp.int32)
                        lo16 = jnp.full((16,), s % 128, dtype=jnp.int32)
                        p0 = plsc.load_gather(pe_v, [hi16, lo16])
                        for j in range(8):
                            for cc in range(8):
                                off = si * _B + j * 128 + cc * 16
                                tok = idx_v[pl.ds(off, 16)]
                                hi = tok >> 7
                                lo = tok & 127
                                vals = plsc.load_gather(row_v, [hi, lo])
                                orow[j, pl.ds(cc * 16, 16)] = vals + p0
                        pltpu.async_copy(orow, spo_dst, osem)
                for q in range(2):
                    pltpu.make_async_copy(
                        orows[q], spo.at[buf, grp_l, 7, :, sub, :],
                        osems[q]).wait()
                plsc.subcore_barrier()

                @pl.when(sub == 0)
                def _drain():
                    dst = out_hbm.at[pl.ds(sg * 8, 8), g]
                    if p == 0:
                        @pl.when(sg >= 1)
                        def _w():
                            pltpu.make_async_copy(
                                spo.at[buf, grp_l], dst, dsem).wait()
                    else:
                        pltpu.make_async_copy(
                            spo.at[buf, grp_l], dst, dsem).wait()
                    pltpu.async_copy(spo.at[buf, grp_l], dst, dsem)

                plsc.subcore_barrier()
                return carry

            lax.fori_loop(0, _SG, sg_body, 0)

        @pl.when(sub == 0)
        def _final_drain():
            pltpu.make_async_copy(spo.at[0, grp_l],
                                  out_hbm.at[pl.ds(0, 8), 0], dsem).wait()

    return k(idx_flat, tp3, pe8)


def kernel(input_x, table):
    idx_flat = input_x.T.reshape(_N).astype(jnp.int32)
    tp3 = jnp.pad(table, ((0, _VP - _V), (0, 0))).T.reshape(_E, 782, 128)
    pe8 = _pe_table_t()
    out5 = _sc_call(idx_flat, tp3, pe8)
    return out5.transpose(2, 4, 0, 1, 3).reshape(_B, _S, _E)


# speedup vs baseline: 1.1958x; 1.1958x over previous
import functools

import jax
import jax.numpy as jnp
from jax import lax
from jax.experimental import pallas as pl
from jax.experimental.pallas import tpu as pltpu
from jax.experimental.pallas import tpu_sc as plsc

_V = 100000
_VP = 782 * 128
_E = 64
_B = 1024
_S = 200
_N = _B * _S

_PASSES = 2
_SG = _S // 8


def _pe_table_t():
    pos = jnp.arange(_S, dtype=jnp.float32)[:, None]
    denom = 10000.0 ** ((jnp.arange(_E) // 2).astype(jnp.float32) / _E)[None, :]
    ang = pos / denom
    pe = jnp.where((jnp.arange(_E) % 2)[None, :] == 0, jnp.sin(ang), jnp.cos(ang))
    return jnp.pad(pe.T, ((0, 0), (0, 1024 - _S))).reshape(_E, 8, 128)


def _sc_call(idx_flat, tp3, pe8):
    mesh = plsc.VectorSubcoreMesh(core_axis_name="c", subcore_axis_name="s")

    @functools.partial(
        pl.kernel,
        mesh=mesh,
        out_type=jax.ShapeDtypeStruct((_S, 8, 8, 8, 128), jnp.float32),
        compiler_params=pltpu.CompilerParams(needs_layout_passes=False),
        scratch_types=[
            pltpu.VMEM((782, 128), jnp.float32),
            pltpu.VMEM((8 * _B,), jnp.int32),
            pltpu.VMEM((8, 128), jnp.float32),
            pltpu.VMEM((8, 8, 128), jnp.float32),
            pltpu.VMEM((8, 8, 128), jnp.float32),
            pltpu.SemaphoreType.DMA,
            pltpu.SemaphoreType.DMA,
        ],
    )
    def k(idx_hbm, tp3_hbm, pe8_hbm, out_hbm, row_v, idx_v, pe_v,
          ow0, ow1, ws0, ws1):
        sid = lax.axis_index("s")
        cid = lax.axis_index("c")
        grp_l = sid // 8
        sub = sid % 8
        obufs = (ow0, ow1)
        wsems = (ws0, ws1)

        for p in range(_PASSES):
            g = 4 * p + 2 * cid + grp_l
            e = 8 * g + sub
            pltpu.sync_copy(tp3_hbm.at[e], row_v)
            pltpu.sync_copy(pe8_hbm.at[e], pe_v)

            def process(sg, half, first_round):
                ob = obufs[half]
                wsem = wsems[half]
                dst = out_hbm.at[pl.ds(sg * 8, 8), g, :, sub, :]
                pltpu.sync_copy(idx_hbm.at[pl.ds(sg * (8 * _B), 8 * _B)],
                                idx_v)
                if first_round:
                    @pl.when(sg >= 2)
                    def _w():
                        pltpu.make_async_copy(ob, dst, wsem).wait()
                else:
                    pltpu.make_async_copy(ob, dst, wsem).wait()

                def si_body(si, c2):
                    s = sg * 8 + si
                    hi16 = jnp.full((16,), s // 128, dtype=jnp.int32)
                    lo16 = jnp.full((16,), s % 128, dtype=jnp.int32)
                    p0 = plsc.load_gather(pe_v, [hi16, lo16])
                    for j in range(8):
                        for cc in range(8):
                            off = si * _B + j * 128 + cc * 16
                            tok = idx_v[pl.ds(off, 16)]
                            vals = plsc.load_gather(row_v,
                                                    [tok >> 7, tok & 127])
                            ob[si, j, pl.ds(cc * 16, 16)] = vals + p0
                    return c2

                lax.fori_loop(0, 8, si_body, 0)
                pltpu.async_copy(ob, dst, wsem)

            def dbl_body(t, carry):
                process(2 * t, 0, p == 0)
                process(2 * t + 1, 1, p == 0)
                return carry

            lax.fori_loop(0, _SG // 2, dbl_body, 0)
            process(jnp.int32(_SG - 1), 0, False)

        dummy = out_hbm.at[pl.ds(0, 8), 0, :, 0, :]
        pltpu.make_async_copy(obufs[0], dummy, wsems[0]).wait()
        pltpu.make_async_copy(obufs[1], dummy, wsems[1]).wait()

    return k(idx_flat, tp3, pe8)


def kernel(input_x, table):
    idx_flat = input_x.T.reshape(_N).astype(jnp.int32)
    tp3 = jnp.pad(table, ((0, _VP - _V), (0, 0))).T.reshape(_E, 782, 128)
    pe8 = _pe_table_t()
    out5 = _sc_call(idx_flat, tp3, pe8)
    return out5.transpose(2, 4, 0, 1, 3).reshape(_B, _S, _E)


# speedup vs baseline: 1.9065x; 1.5943x over previous
import functools

import jax
import jax.numpy as jnp
from jax import lax
from jax.experimental import pallas as pl
from jax.experimental.pallas import tpu as pltpu
from jax.experimental.pallas import tpu_sc as plsc

_V = 100000
_VP = 782 * 128
_E = 64
_B = 1024
_S = 200
_N = _B * _S

_PASSES = 2
_SG = _S // 8


def _pe_table_t():
    pos = jnp.arange(_S, dtype=jnp.float32)[:, None]
    denom = 10000.0 ** ((jnp.arange(_E) // 2).astype(jnp.float32) / _E)[None, :]
    ang = pos / denom
    pe = jnp.where((jnp.arange(_E) % 2)[None, :] == 0, jnp.sin(ang), jnp.cos(ang))
    return jnp.pad(pe.T, ((0, 0), (0, 1024 - _S))).reshape(_E, 8, 128)


def _sc_call(idx_flat, tp3, pe8):
    mesh = plsc.VectorSubcoreMesh(core_axis_name="c", subcore_axis_name="s")

    @functools.partial(
        pl.kernel,
        mesh=mesh,
        out_type=jax.ShapeDtypeStruct((_S, 8, 8, 8, 128), jnp.float32),
        compiler_params=pltpu.CompilerParams(needs_layout_passes=False),
        scratch_types=[
            pltpu.VMEM((782, 128), jnp.float32),
            pltpu.VMEM((8 * _B,), jnp.int32),
            pltpu.VMEM((8, 128), jnp.float32),
            pltpu.VMEM((8, 8, 128), jnp.float32),
            pltpu.VMEM((8, 8, 128), jnp.float32),
            pltpu.SemaphoreType.DMA,
            pltpu.SemaphoreType.DMA,
        ],
    )
    def k(idx_hbm, tp3_hbm, pe8_hbm, out_hbm, row_v, idx_v, pe_v,
          ow0, ow1, ws0, ws1):
        sid = lax.axis_index("s")
        cid = lax.axis_index("c")
        grp_l = sid // 8
        sub = sid % 8
        obufs = (ow0, ow1)
        wsems = (ws0, ws1)

        for p in range(_PASSES):
            g = 4 * p + 2 * cid + grp_l
            e = 8 * g + sub
            pltpu.sync_copy(tp3_hbm.at[e], row_v)
            pltpu.sync_copy(pe8_hbm.at[e], pe_v)

            def process(sg, half, first_round):
                ob = obufs[half]
                wsem = wsems[half]
                dst = out_hbm.at[pl.ds(sg * 8, 8), g, :, sub, :]
                pltpu.sync_copy(idx_hbm.at[pl.ds(sg * (8 * _B), 8 * _B)],
                                idx_v)
                if first_round:
                    @pl.when(sg >= 2)
                    def _w():
                        pltpu.make_async_copy(ob, dst, wsem).wait()
                else:
                    pltpu.make_async_copy(ob, dst, wsem).wait()

                zero16 = jnp.zeros((16,), dtype=jnp.int32)
                for si in range(8):
                    s = sg * 8 + si
                    hi16 = jnp.full((16,), s // 128, dtype=jnp.int32)
                    lo16 = jnp.full((16,), s % 128, dtype=jnp.int32)
                    p0 = plsc.load_gather(pe_v, [hi16, lo16])

                    @plsc.parallel_loop(0, 64, 1, unroll=8)
                    def _chunk(k, si=si, p0=p0):
                        tok = idx_v[pl.ds(si * _B + k * 16, 16)]
                        vals = plsc.load_gather(row_v, [zero16, tok])
                        ob[si, k >> 3, pl.ds((k & 7) * 16, 16)] = vals + p0
                pltpu.async_copy(ob, dst, wsem)

            def dbl_body(t, carry):
                process(2 * t, 0, p == 0)
                process(2 * t + 1, 1, p == 0)
                return carry

            lax.fori_loop(0, _SG // 2, dbl_body, 0)
            process(jnp.int32(_SG - 1), 0, False)

        dummy = out_hbm.at[pl.ds(0, 8), 0, :, 0, :]
        pltpu.make_async_copy(obufs[0], dummy, wsems[0]).wait()
        pltpu.make_async_copy(obufs[1], dummy, wsems[1]).wait()

    return k(idx_flat, tp3, pe8)


def kernel(input_x, table):
    idx_flat = input_x.T.reshape(_N).astype(jnp.int32)
    tp3 = jnp.pad(table, ((0, _VP - _V), (0, 0))).T.reshape(_E, 782, 128)
    pe8 = _pe_table_t()
    out5 = _sc_call(idx_flat, tp3, pe8)
    return out5.transpose(2, 4, 0, 1, 3).reshape(_B, _S, _E)


# speedup vs baseline: 2.1749x; 1.1408x over previous
import functools

import jax
import jax.numpy as jnp
from jax import lax
from jax.experimental import pallas as pl
from jax.experimental.pallas import tpu as pltpu
from jax.experimental.pallas import tpu_sc as plsc

_V = 100000
_VP = 782 * 128
_E = 64
_B = 1024
_S = 200
_N = _B * _S

_PASSES = 2
_SG = _S // 8


def _pe_table_t():
    pos = jnp.arange(_S, dtype=jnp.float32)[:, None]
    denom = 10000.0 ** ((jnp.arange(_E) // 2).astype(jnp.float32) / _E)[None, :]
    ang = pos / denom
    pe = jnp.where((jnp.arange(_E) % 2)[None, :] == 0, jnp.sin(ang), jnp.cos(ang))
    return jnp.pad(pe.T, ((0, 0), (0, 1024 - _S))).reshape(_E, 8, 128)


def _sc_call(idx_flat, tp3, pe8):
    mesh = plsc.VectorSubcoreMesh(core_axis_name="c", subcore_axis_name="s")

    @functools.partial(
        pl.kernel,
        mesh=mesh,
        out_type=jax.ShapeDtypeStruct((_S, 8, 8, 8, 128), jnp.float32),
        compiler_params=pltpu.CompilerParams(needs_layout_passes=False),
        scratch_types=[
            pltpu.VMEM((782, 128), jnp.float32),
            pltpu.VMEM((4 * _B,), jnp.int32),
            pltpu.VMEM((4 * _B,), jnp.int32),
            pltpu.VMEM((8, 128), jnp.float32),
            pltpu.VMEM((4, 8, 128), jnp.float32),
            pltpu.VMEM((4, 8, 128), jnp.float32),
            pltpu.SemaphoreType.DMA,
            pltpu.SemaphoreType.DMA,
            pltpu.SemaphoreType.DMA,
            pltpu.SemaphoreType.DMA,
        ],
    )
    def k(idx_hbm, tp3_hbm, pe8_hbm, out_hbm, row_v, iv0, iv1, pe_v,
          ow0, ow1, ws0, ws1, is0, is1):
        sid = lax.axis_index("s")
        cid = lax.axis_index("c")
        grp_l = sid // 8
        sub = sid % 8
        obufs = (ow0, ow1)
        wsems = (ws0, ws1)
        ivs = (iv0, iv1)
        isems = (is0, is1)
        nslab = 2 * _SG

        def slab_src(sg):
            return idx_hbm.at[pl.ds(sg * (4 * _B), 4 * _B)]

        for p in range(_PASSES):
            g = 4 * p + 2 * cid + grp_l
            e = 8 * g + sub
            pltpu.sync_copy(tp3_hbm.at[e], row_v)
            pltpu.sync_copy(pe8_hbm.at[e], pe_v)
            pltpu.async_copy(slab_src(0), iv0, is0)
            pltpu.async_copy(slab_src(jnp.int32(1)), iv1, is1)

            def process(sg, t, half, first_round):
                ob = obufs[half]
                wsem = wsems[half]
                iv = ivs[half]
                isem = isems[half]
                dst = out_hbm.at[pl.ds(sg * 4, 4), g, :, sub, :]
                pltpu.make_async_copy(slab_src(sg), iv, isem).wait()
                if first_round:
                    @pl.when(t >= 1)
                    def _w():
                        pltpu.make_async_copy(ob, dst, wsem).wait()
                else:
                    pltpu.make_async_copy(ob, dst, wsem).wait()

                zero16 = jnp.zeros((16,), dtype=jnp.int32)
                for si in range(4):
                    s = sg * 4 + si
                    hi16 = jnp.full((16,), s // 128, dtype=jnp.int32)
                    lo16 = jnp.full((16,), s % 128, dtype=jnp.int32)
                    p0 = plsc.load_gather(pe_v, [hi16, lo16])

                    @plsc.parallel_loop(0, 64, 1, unroll=16)
                    def _chunk(k, si=si, p0=p0):
                        tok = iv[pl.ds(si * _B + k * 16, 16)]
                        vals = plsc.load_gather(row_v, [zero16, tok])
                        ob[si, k >> 3, pl.ds((k & 7) * 16, 16)] = vals + p0
                pltpu.async_copy(ob, dst, wsem)

                @pl.when(sg + 2 < nslab)
                def _pf():
                    pltpu.async_copy(slab_src(sg + 2), iv, isem)

            def dbl_body(t, carry):
                process(2 * t, t, 0, p == 0)
                process(2 * t + 1, t, 1, p == 0)
                return carry

            lax.fori_loop(0, nslab // 2, dbl_body, 0)

        dummy = out_hbm.at[pl.ds(0, 4), 0, :, 0, :]
        pltpu.make_async_copy(obufs[0], dummy, wsems[0]).wait()
        pltpu.make_async_copy(obufs[1], dummy, wsems[1]).wait()

    return k(idx_flat, tp3, pe8)


def kernel(input_x, table):
    idx_flat = input_x.T.reshape(_N).astype(jnp.int32)
    tp3 = jnp.pad(table, ((0, _VP - _V), (0, 0))).T.reshape(_E, 782, 128)
    pe8 = _pe_table_t()
    out5 = _sc_call(idx_flat, tp3, pe8)
    return out5.transpose(2, 4, 0, 1, 3).reshape(_B, _S, _E)


# speedup vs baseline: 2.3555x; 1.0830x over previous
import functools

import jax
import jax.numpy as jnp
from jax import lax
from jax.experimental import pallas as pl
from jax.experimental.pallas import tpu as pltpu
from jax.experimental.pallas import tpu_sc as plsc

_V = 100000
_VP = 782 * 128
_E = 64
_B = 1024
_S = 200
_N = _B * _S

_PASSES = 2
_SG = _S // 8


def _pe_table_t():
    pos = jnp.arange(_S, dtype=jnp.float32)[:, None]
    denom = 10000.0 ** ((jnp.arange(_E) // 2).astype(jnp.float32) / _E)[None, :]
    ang = pos / denom
    pe = jnp.where((jnp.arange(_E) % 2)[None, :] == 0, jnp.sin(ang), jnp.cos(ang))
    return jnp.pad(pe.T, ((0, 0), (0, 1024 - _S))).reshape(_E, 8, 128)


def _sc_call(idx_flat, tp3, pe8):
    mesh = plsc.VectorSubcoreMesh(core_axis_name="c", subcore_axis_name="s")

    @functools.partial(
        pl.kernel,
        mesh=mesh,
        out_type=jax.ShapeDtypeStruct((_S, 8, 8, 8, 128), jnp.float32),
        compiler_params=pltpu.CompilerParams(needs_layout_passes=False),
        scratch_types=[
            pltpu.VMEM((782, 128), jnp.float32),
            pltpu.VMEM((8, 8, 128), jnp.int32),
            pltpu.VMEM((8, 8, 128), jnp.int32),
            pltpu.VMEM((8, 128), jnp.float32),
            pltpu.VMEM((4, 8, 128), jnp.float32),
            pltpu.VMEM((4, 8, 128), jnp.float32),
            pltpu.SemaphoreType.DMA,
            pltpu.SemaphoreType.DMA,
            pltpu.SemaphoreType.DMA,
            pltpu.SemaphoreType.DMA,
        ],
    )
    def k(idx_hbm, tp3_hbm, pe8_hbm, out_hbm, row_v, iv0, iv1, pe_v,
          ow0, ow1, ws0, ws1, is0, is1):
        sid = lax.axis_index("s")
        cid = lax.axis_index("c")
        grp_l = sid // 8
        sub = sid % 8
        obufs = (ow0, ow1)
        wsems = (ws0, ws1)
        ivs = (iv0, iv1)
        isems = (is0, is1)

        for p in range(_PASSES):
            g = 4 * p + 2 * cid + grp_l
            e = 8 * g + sub
            pltpu.sync_copy(tp3_hbm.at[e], row_v)
            pltpu.sync_copy(pe8_hbm.at[e], pe_v)
            pltpu.async_copy(idx_hbm.at[0], iv0, is0)
            pltpu.async_copy(idx_hbm.at[jnp.int32(1)], iv1, is1)

            def process(st, half, first_round):
                ob0 = obufs[half]
                ob1 = obufs[1 - half]
                iv = ivs[half]
                isem = isems[half]
                pltpu.make_async_copy(idx_hbm.at[st], iv, isem).wait()
                zero16 = jnp.zeros((16,), dtype=jnp.int32)
                for hh in range(2):
                    ob = (ob0, ob1)[hh]
                    wsem = (wsems[half], wsems[1 - half])[hh]
                    dst = out_hbm.at[pl.ds(st * 8 + hh * 4, 4), g, :, sub, :]
                    if first_round:
                        @pl.when(st >= 1)
                        def _w():
                            pltpu.make_async_copy(ob, dst, wsem).wait()
                    else:
                        pltpu.make_async_copy(ob, dst, wsem).wait()
                    for si in range(4):
                        s = st * 8 + hh * 4 + si
                        hi16 = jnp.full((16,), s // 128, dtype=jnp.int32)
                        lo16 = jnp.full((16,), s % 128, dtype=jnp.int32)
                        p0 = plsc.load_gather(pe_v, [hi16, lo16])

                        @plsc.parallel_loop(0, 64, 1, unroll=16)
                        def _chunk(k, hh=hh, si=si, p0=p0, ob=ob, iv=iv):
                            tok = iv[k >> 3, hh * 4 + si,
                                     pl.ds((k & 7) * 16, 16)]
                            vals = plsc.load_gather(row_v, [zero16, tok])
                            ob[si, k >> 3, pl.ds((k & 7) * 16, 16)] = \
                                vals + p0
                    pltpu.async_copy(ob, dst, wsem)

                @pl.when(st + 2 < _SG)
                def _pf():
                    pltpu.async_copy(idx_hbm.at[st + 2], iv, isem)

            def dbl_body(t, carry):
                process(2 * t, 0, p == 0)
                process(2 * t + 1, 1, p == 0)
                return carry

            lax.fori_loop(0, _SG // 2, dbl_body, 0)
            process(jnp.int32(_SG - 1), 0, False)

        dummy = out_hbm.at[pl.ds(0, 4), 0, :, 0, :]
        pltpu.make_async_copy(obufs[0], dummy, wsems[0]).wait()
        pltpu.make_async_copy(obufs[1], dummy, wsems[1]).wait()

    return k(idx_flat, tp3, pe8)


def kernel(input_x, table):
    idx4 = (input_x.astype(jnp.int32)
            .reshape(8, 128, 25, 8).transpose(2, 0, 3, 1))
    tp3 = jnp.pad(table, ((0, _VP - _V), (0, 0))).T.reshape(_E, 782, 128)
    pe8 = _pe_table_t()
    out5 = _sc_call(idx4, tp3, pe8)
    return out5.transpose(2, 4, 0, 1, 3).reshape(_B, _S, _E)


# speedup vs baseline: 2.5708x; 1.0914x over previous
import functools

import jax
import jax.numpy as jnp
from jax import lax
from jax.experimental import pallas as pl
from jax.experimental.pallas import tpu as pltpu
from jax.experimental.pallas import tpu_sc as plsc

_V = 100000
_VP = 782 * 128
_E = 64
_B = 1024
_S = 200
_N = _B * _S

_PASSES = 2
_SG = _S // 8


def _pe_table_t():
    pos = jnp.arange(_S, dtype=jnp.float32)[:, None]
    denom = 10000.0 ** ((jnp.arange(_E) // 2).astype(jnp.float32) / _E)[None, :]
    ang = pos / denom
    pe = jnp.where((jnp.arange(_E) % 2)[None, :] == 0, jnp.sin(ang), jnp.cos(ang))
    return jnp.pad(pe.T, ((0, 0), (0, 1024 - _S))).reshape(_E, 8, 128)


def _sc_call(idx_flat, tp3, pe8):
    mesh = plsc.VectorSubcoreMesh(core_axis_name="c", subcore_axis_name="s")

    @functools.partial(
        pl.kernel,
        mesh=mesh,
        out_type=jax.ShapeDtypeStruct((_S, 8, 8, 8, 128), jnp.float32),
        compiler_params=pltpu.CompilerParams(needs_layout_passes=False),
        scratch_types=[
            pltpu.VMEM_SHARED((_SG, 8, 8, 128), jnp.int32),
            pltpu.VMEM((782, 128), jnp.float32),
            pltpu.VMEM((8, 8, 128), jnp.int32),
            pltpu.VMEM((8, 128), jnp.float32),
            pltpu.VMEM((4, 8, 128), jnp.float32),
            pltpu.VMEM((4, 8, 128), jnp.float32),
            pltpu.SemaphoreType.DMA,
            pltpu.SemaphoreType.DMA,
        ],
    )
    def k(idx_hbm, tp3_hbm, pe8_hbm, out_hbm, idx_sp, row_v, iv, pe_v,
          ow0, ow1, ws0, ws1):
        sid = lax.axis_index("s")
        cid = lax.axis_index("c")
        grp_l = sid // 8
        sub = sid % 8
        obufs = (ow0, ow1)
        wsems = (ws0, ws1)

        @pl.when(sid == 0)
        def _stage_idx():
            pltpu.sync_copy(idx_hbm, idx_sp)

        plsc.subcore_barrier()

        for p in range(_PASSES):
            g = 4 * p + 2 * cid + grp_l
            e = 8 * g + sub
            pltpu.sync_copy(tp3_hbm.at[e], row_v)
            pltpu.sync_copy(pe8_hbm.at[e], pe_v)

            def process(st, half, first_round):
                ob0 = obufs[half]
                ob1 = obufs[1 - half]
                pltpu.sync_copy(idx_sp.at[st], iv)
                zero16 = jnp.zeros((16,), dtype=jnp.int32)
                for hh in range(2):
                    ob = (ob0, ob1)[hh]
                    wsem = (wsems[half], wsems[1 - half])[hh]
                    dst = out_hbm.at[pl.ds(st * 8 + hh * 4, 4), g, :, sub, :]
                    if first_round:
                        @pl.when(st >= 1)
                        def _w():
                            pltpu.make_async_copy(ob, dst, wsem).wait()
                    else:
                        pltpu.make_async_copy(ob, dst, wsem).wait()
                    for si in range(4):
                        s = st * 8 + hh * 4 + si
                        hi16 = jnp.full((16,), s // 128, dtype=jnp.int32)
                        lo16 = jnp.full((16,), s % 128, dtype=jnp.int32)
                        p0 = plsc.load_gather(pe_v, [hi16, lo16])

                        @plsc.parallel_loop(0, 64, 1, unroll=32)
                        def _chunk(k, hh=hh, si=si, p0=p0, ob=ob, iv=iv):
                            tok = iv[k >> 3, hh * 4 + si,
                                     pl.ds((k & 7) * 16, 16)]
                            vals = plsc.load_gather(row_v, [zero16, tok])
                            ob[si, k >> 3, pl.ds((k & 7) * 16, 16)] = \
                                vals + p0
                    pltpu.async_copy(ob, dst, wsem)

            def dbl_body(t, carry):
                process(2 * t, 0, p == 0)
                process(2 * t + 1, 1, p == 0)
                return carry

            lax.fori_loop(0, _SG // 2, dbl_body, 0)
            process(jnp.int32(_SG - 1), 0, False)

        dummy = out_hbm.at[pl.ds(0, 4), 0, :, 0, :]
        pltpu.make_async_copy(obufs[0], dummy, wsems[0]).wait()
        pltpu.make_async_copy(obufs[1], dummy, wsems[1]).wait()

    return k(idx_flat, tp3, pe8)


def kernel(input_x, table):
    idx4 = (input_x.astype(jnp.int32)
            .reshape(8, 128, 25, 8).transpose(2, 0, 3, 1))
    tp3 = jnp.pad(table, ((0, _VP - _V), (0, 0))).T.reshape(_E, 782, 128)
    pe8 = _pe_table_t()
    out5 = _sc_call(idx4, tp3, pe8)
    return out5.transpose(2, 4, 0, 1, 3).reshape(_B, _S, _E)


# speedup vs baseline: 3.0817x; 1.1987x over previous
import functools

import jax
import jax.numpy as jnp
from jax import lax
from jax.experimental import pallas as pl
from jax.experimental.pallas import tpu as pltpu
from jax.experimental.pallas import tpu_sc as plsc

_V = 100000
_VP = 782 * 128
_E = 64
_B = 1024
_S = 200
_N = _B * _S

_PASSES = 2
_SG = _S // 8


def _pe_table_t():
    pos = jnp.arange(_S, dtype=jnp.float32)[:, None]
    denom = 10000.0 ** ((jnp.arange(_E) // 2).astype(jnp.float32) / _E)[None, :]
    ang = pos / denom
    pe = jnp.where((jnp.arange(_E) % 2)[None, :] == 0, jnp.sin(ang), jnp.cos(ang))
    return jnp.pad(pe.T, ((0, 0), (0, 1024 - _S))).reshape(_E, 8, 128)


def _sc_call(idx_flat, tp3, pe8):
    mesh = plsc.VectorSubcoreMesh(core_axis_name="c", subcore_axis_name="s")

    @functools.partial(
        pl.kernel,
        mesh=mesh,
        out_type=jax.ShapeDtypeStruct((_S, 8, 8, 8, 128), jnp.float32),
        compiler_params=pltpu.CompilerParams(needs_layout_passes=False),
        scratch_types=[
            pltpu.VMEM_SHARED((_SG, 8, 8, 128), jnp.int32),
            pltpu.VMEM((782, 128), jnp.float32),
            pltpu.VMEM((8, 4, 128), jnp.int32),
            pltpu.VMEM((8, 4, 128), jnp.int32),
            pltpu.VMEM((8, 128), jnp.float32),
            pltpu.VMEM((4, 8, 128), jnp.float32),
            pltpu.VMEM((4, 8, 128), jnp.float32),
            pltpu.SemaphoreType.DMA,
            pltpu.SemaphoreType.DMA,
            pltpu.SemaphoreType.DMA,
            pltpu.SemaphoreType.DMA,
        ],
    )
    def k(idx_hbm, tp3_hbm, pe8_hbm, out_hbm, idx_sp, row_v, iv0, iv1, pe_v,
          ow0, ow1, ws0, ws1, is0, is1):
        sid = lax.axis_index("s")
        cid = lax.axis_index("c")
        grp_l = sid // 8
        sub = sid % 8
        obufs = (ow0, ow1)
        wsems = (ws0, ws1)
        ivs = (iv0, iv1)
        isems = (is0, is1)

        def unit_src(st, hh):
            return idx_sp.at[st, :, pl.ds(hh * 4, 4), :]

        @pl.when(sid == 0)
        def _stage_idx():
            pltpu.sync_copy(idx_hbm, idx_sp)

        plsc.subcore_barrier()

        for p in range(_PASSES):
            g = 4 * p + 2 * cid + grp_l
            e = 8 * g + sub
            pltpu.sync_copy(tp3_hbm.at[e], row_v)
            pltpu.sync_copy(pe8_hbm.at[e], pe_v)
            pltpu.async_copy(unit_src(0, 0), iv0, is0)
            pltpu.async_copy(unit_src(0, 1), iv1, is1)

            def half(st, hh, first_round):
                iv = ivs[hh]
                isem = isems[hh]
                ob = obufs[hh]
                wsem = wsems[hh]
                dst = out_hbm.at[pl.ds(st * 8 + hh * 4, 4), g, :, sub, :]
                pltpu.make_async_copy(unit_src(st, hh), iv, isem).wait()
                if first_round:
                    @pl.when(st >= 1)
                    def _w():
                        pltpu.make_async_copy(ob, dst, wsem).wait()
                else:
                    pltpu.make_async_copy(ob, dst, wsem).wait()
                zero16 = jnp.zeros((16,), dtype=jnp.int32)
                for si in range(4):
                    s = st * 8 + hh * 4 + si
                    hi16 = jnp.full((16,), s // 128, dtype=jnp.int32)
                    lo16 = jnp.full((16,), s % 128, dtype=jnp.int32)
                    p0 = plsc.load_gather(pe_v, [hi16, lo16])

                    @plsc.parallel_loop(0, 64, 1, unroll=32)
                    def _chunk(k, si=si, p0=p0, ob=ob, iv=iv):
                        tok = iv[k >> 3, si, pl.ds((k & 7) * 16, 16)]
                        vals = plsc.load_gather(row_v, [zero16, tok])
                        ob[si, k >> 3, pl.ds((k & 7) * 16, 16)] = vals + p0
                pltpu.async_copy(ob, dst, wsem)

                @pl.when(st + 1 < _SG)
                def _pf():
                    pltpu.async_copy(unit_src(st + 1, hh), iv, isem)

            def st_body(st, carry):
                half(st, 0, p == 0)
                half(st, 1, p == 0)
                return carry

            lax.fori_loop(0, _SG, st_body, 0)

        dummy = out_hbm.at[pl.ds(0, 4), 0, :, 0, :]
        pltpu.make_async_copy(obufs[0], dummy, wsems[0]).wait()
        pltpu.make_async_copy(obufs[1], dummy, wsems[1]).wait()

    return k(idx_flat, tp3, pe8)


def kernel(input_x, table):
    idx4 = (input_x.astype(jnp.int32)
            .reshape(8, 128, 25, 8).transpose(2, 0, 3, 1))
    tp3 = jnp.pad(table, ((0, _VP - _V), (0, 0))).T.reshape(_E, 782, 128)
    pe8 = _pe_table_t()
    out5 = _sc_call(idx4, tp3, pe8)
    return out5.transpose(2, 4, 0, 1, 3).reshape(_B, _S, _E)


# speedup vs baseline: 3.1805x; 1.0321x over previous
import functools

import jax
import jax.numpy as jnp
from jax import lax
from jax.experimental import pallas as pl
from jax.experimental.pallas import tpu as pltpu
from jax.experimental.pallas import tpu_sc as plsc

_V = 100000
_VP = 782 * 128
_E = 64
_B = 1024
_S = 200
_N = _B * _S

_PASSES = 2
_SG = _S // 8


def _pe_table_t():
    pos = jnp.arange(_S, dtype=jnp.float32)[:, None]
    denom = 10000.0 ** ((jnp.arange(_E) // 2).astype(jnp.float32) / _E)[None, :]
    ang = pos / denom
    pe = jnp.where((jnp.arange(_E) % 2)[None, :] == 0, jnp.sin(ang), jnp.cos(ang))
    return jnp.pad(pe.T, ((0, 0), (0, 1024 - _S))).reshape(_E, 8, 128)


def _sc_call(idx_flat, tp3, pe8):
    mesh = plsc.VectorSubcoreMesh(core_axis_name="c", subcore_axis_name="s")

    @functools.partial(
        pl.kernel,
        mesh=mesh,
        out_type=jax.ShapeDtypeStruct((_S, 8, 8, 8, 128), jnp.float32),
        compiler_params=pltpu.CompilerParams(needs_layout_passes=False),
        scratch_types=[
            pltpu.VMEM_SHARED((_SG, 8, 8, 128), jnp.int32),
            pltpu.VMEM((782, 128), jnp.float32),
            pltpu.VMEM((8, 4, 128), jnp.int32),
            pltpu.VMEM((8, 4, 128), jnp.int32),
            pltpu.VMEM((8, 128), jnp.float32),
            pltpu.VMEM((4, 8, 128), jnp.float32),
            pltpu.VMEM((4, 8, 128), jnp.float32),
            pltpu.SemaphoreType.DMA,
            pltpu.SemaphoreType.DMA,
            pltpu.SemaphoreType.DMA,
            pltpu.SemaphoreType.DMA,
        ],
    )
    def k(idx_hbm, tp3_hbm, pe8_hbm, out_hbm, idx_sp, row_v, iv0, iv1, pe_v,
          ow0, ow1, ws0, ws1, is0, is1):
        sid = lax.axis_index("s")
        cid = lax.axis_index("c")
        grp_l = sid // 8
        sub = sid % 8
        obufs = (ow0, ow1)
        wsems = (ws0, ws1)
        ivs = (iv0, iv1)
        isems = (is0, is1)

        def unit_src(st, hh):
            return idx_sp.at[st, :, pl.ds(hh * 4, 4), :]

        @pl.when(sid == 0)
        def _stage_idx():
            pltpu.sync_copy(idx_hbm, idx_sp)

        plsc.subcore_barrier()

        for p in range(_PASSES):
            g = 4 * p + 2 * cid + grp_l
            e = 8 * g + sub
            pltpu.sync_copy(tp3_hbm.at[e], row_v)
            pltpu.sync_copy(pe8_hbm.at[e], pe_v)
            pltpu.async_copy(unit_src(0, 0), iv0, is0)
            pltpu.async_copy(unit_src(0, 1), iv1, is1)

            def half(st, hh, first_round):
                iv = ivs[hh]
                isem = isems[hh]
                ob = obufs[hh]
                wsem = wsems[hh]
                dst = out_hbm.at[pl.ds(st * 8 + hh * 4, 4), g, :, sub, :]
                pltpu.make_async_copy(unit_src(st, hh), iv, isem).wait()
                if first_round:
                    @pl.when(st >= 1)
                    def _w():
                        pltpu.make_async_copy(ob, dst, wsem).wait()
                else:
                    pltpu.make_async_copy(ob, dst, wsem).wait()
                zero16 = jnp.zeros((16,), dtype=jnp.int32)
                for si in range(4):
                    s = st * 8 + hh * 4 + si
                    hi16 = jnp.full((16,), s // 128, dtype=jnp.int32)
                    lo16 = jnp.full((16,), s % 128, dtype=jnp.int32)
                    p0 = plsc.load_gather(pe_v, [hi16, lo16])

                    @plsc.parallel_loop(0, 64, 1, unroll=64)
                    def _chunk(k, si=si, p0=p0, ob=ob, iv=iv):
                        tok = iv[k >> 3, si, pl.ds((k & 7) * 16, 16)]
                        vals = plsc.load_gather(row_v, [zero16, tok])
                        ob[si, k >> 3, pl.ds((k & 7) * 16, 16)] = vals + p0
                pltpu.async_copy(ob, dst, wsem)

                @pl.when(st + 1 < _SG)
                def _pf():
                    pltpu.async_copy(unit_src(st + 1, hh), iv, isem)

            def st_body(st, carry):
                half(st, 0, p == 0)
                half(st, 1, p == 0)
                return carry

            lax.fori_loop(0, _SG, st_body, 0)

        dummy = out_hbm.at[pl.ds(0, 4), 0, :, 0, :]
        pltpu.make_async_copy(obufs[0], dummy, wsems[0]).wait()
        pltpu.make_async_copy(obufs[1], dummy, wsems[1]).wait()

    return k(idx_flat, tp3, pe8)


def kernel(input_x, table):
    idx4 = (input_x.astype(jnp.int32)
            .reshape(8, 128, 25, 8).transpose(2, 0, 3, 1))
    tp3 = jnp.pad(table, ((0, _VP - _V), (0, 0))).T.reshape(_E, 782, 128)
    pe8 = _pe_table_t()
    out5 = _sc_call(idx4, tp3, pe8)
    return out5.transpose(2, 4, 0, 1, 3).reshape(_B, _S, _E)
